# baseline (speedup 1.0000x reference)
import functools
import jax
import jax.numpy as jnp
from jax import lax
from jax.experimental import pallas as pl
from jax.experimental.pallas import tpu as pltpu
from jax.experimental.pallas import tpu_sc as plsc

BATCH = 16384
FEAT = 128
SHAPE_DIM = 300
OUT_DIM = SHAPE_DIM + FEAT
ROW_PAD = 384

NC = 2
NS = 16
NW = NC * NS
B_PER_W = BATCH // NW
CHUNK = 128
NCHUNK = B_PER_W // CHUNK

_mesh = plsc.VectorSubcoreMesh(core_axis_name="c", subcore_axis_name="s")


@functools.partial(
    pl.kernel,
    mesh=_mesh,
    out_type=jax.ShapeDtypeStruct((OUT_DIM, BATCH), jnp.float32),
    compiler_params=pltpu.CompilerParams(
        needs_layout_passes=False,
        skip_device_barrier=True,
        disable_bounds_checks=True,
        disable_semaphore_checks=True,
    ),
    scratch_types=[
        pltpu.VMEM((CHUNK,), jnp.int32),
        pltpu.VMEM((CHUNK, ROW_PAD), jnp.float32),
        pltpu.VMEM((CHUNK, FEAT), jnp.float32),
        pltpu.VMEM((OUT_DIM, CHUNK), jnp.float32),
        pltpu.SemaphoreType.DMA,
        pltpu.SemaphoreType.DMA,
        pltpu.SemaphoreType.DMA,
    ],
)
def _concat_shape_t(inputs_hbm, idx_hbm, table_hbm, outT_hbm,
                    idx_v, buf_v, inp_v, stg_v, sem_g, sem_i, sem_o):
    wid = lax.axis_index("s") * NC + lax.axis_index("c")
    rows = [lax.iota(jnp.int32, 16) + 16 * k for k in range(8)]

    def transpose_feat(src_ref, src_col, dst_row):
        cols = jnp.broadcast_to(jnp.int32(0) + src_col, (16,))
        for k in range(8):
            v = plsc.load_gather(src_ref, [rows[k], cols])
            stg_v[dst_row, pl.ds(16 * k, 16)] = v

    oh = None
    for q in range(NCHUNK):
        base = wid * B_PER_W + q * CHUNK
        pltpu.sync_copy(idx_hbm.at[pl.ds(base, CHUNK)], idx_v)
        gh = pltpu.async_copy(
            table_hbm.at[idx_v, pl.ds(0, ROW_PAD)], buf_v, sem_g)
        ih = pltpu.async_copy(
            inputs_hbm.at[pl.ds(base, CHUNK)], inp_v, sem_i)
        gh.wait()
        ih.wait()
        if oh is not None:
            oh.wait()

        def body_gather(jj, carry):
            for t in range(8):
                transpose_feat(buf_v, 8 * jj + t, 8 * jj + t)
            return carry
        lax.fori_loop(0, 37, body_gather, 0)
        for t in range(4):
            transpose_feat(buf_v, 296 + t, 296 + t)
        for t in range(4):
            transpose_feat(inp_v, t, 300 + t)
        def body_inp(jj, carry):
            for t in range(8):
                transpose_feat(inp_v, 8 * jj + t + 4, 8 * jj + t + 304)
            return carry
        lax.fori_loop(0, 15, body_inp, 0)
        for t in range(4):
            transpose_feat(inp_v, 124 + t, 424 + t)

        oh = pltpu.async_copy(
            stg_v, outT_hbm.at[:, pl.ds(base, CHUNK)], sem_o)
    oh.wait()


def kernel(inputs, subject_id, s):
    return _concat_shape_t(inputs, subject_id.astype(jnp.int32), s).T


# speedup vs baseline: 1.4779x; 1.4779x over previous
import functools
import jax
import jax.numpy as jnp
from jax import lax
from jax.experimental import pallas as pl
from jax.experimental.pallas import tpu as pltpu
from jax.experimental.pallas import tpu_sc as plsc

BATCH = 16384
FEAT = 128
SHAPE_DIM = 300
OUT_DIM = SHAPE_DIM + FEAT
ROW_PAD = 384

NC = 2
NS = 16
NW = NC * NS
B_PER_W = BATCH // NW
CHUNK = 128
NCHUNK = B_PER_W // CHUNK

_mesh = plsc.VectorSubcoreMesh(core_axis_name="c", subcore_axis_name="s")


def _rotate4(v):
    idx = (lax.iota(jnp.int32, 16) + 4) % 16
    return lax.gather(
        v, idx[:, None],
        dimension_numbers=lax.GatherDimensionNumbers(
            offset_dims=(), collapsed_slice_dims=(0,), start_index_map=(0,)),
        slice_sizes=(1,),
        mode=lax.GatherScatterMode.PROMISE_IN_BOUNDS)


@functools.partial(
    pl.kernel,
    mesh=_mesh,
    out_type=jax.ShapeDtypeStruct((OUT_DIM, BATCH), jnp.float32),
    compiler_params=pltpu.CompilerParams(
        needs_layout_passes=False,
        skip_device_barrier=True,
        disable_bounds_checks=True,
        disable_semaphore_checks=True,
    ),
    scratch_types=[
        pltpu.VMEM((CHUNK,), jnp.int32),
        pltpu.VMEM((CHUNK, ROW_PAD), jnp.float32),
        pltpu.VMEM((CHUNK, FEAT), jnp.float32),
        pltpu.VMEM((OUT_DIM, CHUNK), jnp.float32),
        pltpu.SemaphoreType.DMA,
        pltpu.SemaphoreType.DMA,
        pltpu.SemaphoreType.DMA,
    ],
)
def _concat_shape_t(inputs_hbm, idx_hbm, table_hbm, outT_hbm,
                    idx_v, buf_v, inp_v, stg_v, sem_g, sem_i, sem_o):
    wid = lax.axis_index("s") * NC + lax.axis_index("c")
    lanes = lax.iota(jnp.int32, 16)
    bvecs = [lanes + 16 * bg for bg in range(8)]
    perms = [(lanes + d) % 16 for d in range(16)]

    def diag_block(src_ref, src_c0, dst_f0):
        def bg_body(bg, carry):
            bvec = lanes + 16 * bg
            for d in range(16):
                fsrc = src_c0 + perms[d]
                fdst = dst_f0 + perms[d]
                v = plsc.load_gather(src_ref, [bvec, fsrc])
                plsc.store_scatter(stg_v, [fdst, bvec], v)
            return carry
        lax.fori_loop(0, 8, bg_body, 0)

    def diag_block_masked(src_ref, src_c0, dst_f0):
        def bg_body(bg, carry):
            bvec = lanes + 16 * bg
            for d in range(16):
                fsrc = src_c0 + perms[d]
                fdst = dst_f0 + perms[d]
                m = fdst < OUT_DIM
                v = plsc.load_gather(src_ref, [bvec, fsrc], mask=m)
                plsc.store_scatter(stg_v, [fdst, bvec], v, mask=m)
            return carry
        lax.fori_loop(0, 8, bg_body, 0)

    oh = None
    for q in range(NCHUNK):
        base = wid * B_PER_W + q * CHUNK
        pltpu.sync_copy(idx_hbm.at[pl.ds(base, CHUNK)], idx_v)
        gh = pltpu.async_copy(
            table_hbm.at[idx_v, pl.ds(0, ROW_PAD)], buf_v, sem_g)
        ih = pltpu.async_copy(
            inputs_hbm.at[pl.ds(base, CHUNK)], inp_v, sem_i)
        gh.wait()
        ih.wait()
        if oh is not None:
            oh.wait()

        def seam(i, carry):
            vg = buf_v[i, pl.ds(288, 16)]
            rot = _rotate4(inp_v[i, pl.ds(0, 16)])
            buf_v[i, pl.ds(288, 16)] = jnp.where(lanes < 12, vg, rot)
            return carry
        lax.fori_loop(0, CHUNK, seam, 0)

        def body_buf(fg, carry):
            diag_block(buf_v, 16 * fg, 16 * fg)
            return carry
        lax.fori_loop(0, 19, body_buf, 0)
        def body_inp(fg, carry):
            diag_block(inp_v, 16 * fg - 300, 16 * fg)
            return carry
        lax.fori_loop(19, 26, body_inp, 0)
        diag_block_masked(inp_v, 116, 416)

        oh = pltpu.async_copy(
            stg_v, outT_hbm.at[:, pl.ds(base, CHUNK)], sem_o)
    oh.wait()


def kernel(inputs, subject_id, s):
    return _concat_shape_t(inputs, subject_id.astype(jnp.int32), s).T


# speedup vs baseline: 1.6440x; 1.1124x over previous
import functools
import jax
import jax.numpy as jnp
from jax import lax
from jax.experimental import pallas as pl
from jax.experimental.pallas import tpu as pltpu
from jax.experimental.pallas import tpu_sc as plsc

BATCH = 16384
FEAT = 128
SHAPE_DIM = 300
OUT_DIM = SHAPE_DIM + FEAT
ROW_PAD = 384
TAIL = OUT_DIM - ROW_PAD
SPLIT = ROW_PAD - SHAPE_DIM

NC = 2
NS = 16
NW = NC * NS
B_PER_W = BATCH // NW
CHUNK = 64
NCHUNK = B_PER_W // CHUNK
NBUF = 2

_mesh = plsc.VectorSubcoreMesh(core_axis_name="c", subcore_axis_name="s")


def _rotate4(v):
    idx = (lax.iota(jnp.int32, 16) + 4) % 16
    return lax.gather(
        v, idx[:, None],
        dimension_numbers=lax.GatherDimensionNumbers(
            offset_dims=(), collapsed_slice_dims=(0,), start_index_map=(0,)),
        slice_sizes=(1,),
        mode=lax.GatherScatterMode.PROMISE_IN_BOUNDS)


@functools.partial(
    pl.kernel,
    mesh=_mesh,
    out_type=jax.ShapeDtypeStruct((BATCH, OUT_DIM), jnp.float32),
    compiler_params=pltpu.CompilerParams(
        skip_device_barrier=True,
        disable_bounds_checks=True,
        disable_semaphore_checks=True,
    ),
    scratch_types=[
        pltpu.VMEM((NBUF, CHUNK), jnp.int32),
        pltpu.VMEM((NBUF, CHUNK, ROW_PAD), jnp.float32),
        pltpu.VMEM((NBUF, CHUNK, FEAT), jnp.float32),
        pltpu.VMEM((NBUF, CHUNK, TAIL), jnp.float32),
        pltpu.SemaphoreType.DMA((NBUF,)),
        pltpu.SemaphoreType.DMA((NBUF,)),
        pltpu.SemaphoreType.DMA((NBUF,)),
        pltpu.SemaphoreType.DMA((NBUF,)),
    ],
)
def _concat_shape(inputs_hbm, idx_hbm, table_hbm, out_hbm,
                  idx_v, buf_v, inp_v, tail_v, sem_g, sem_i, sem_o, sem_t):
    wid = lax.axis_index("s") * NC + lax.axis_index("c")
    lanes = lax.iota(jnp.int32, 16)

    gh = [None] * NBUF
    ih = [None] * NBUF
    oh = [None] * NBUF
    th = [None] * NBUF

    def start_chunk(c):
        b = c % NBUF
        base = wid * B_PER_W + c * CHUNK
        pltpu.sync_copy(idx_hbm.at[pl.ds(base, CHUNK)], idx_v.at[b])
        gh[b] = pltpu.async_copy(
            table_hbm.at[idx_v.at[b], pl.ds(0, ROW_PAD)],
            buf_v.at[b], sem_g.at[b])
        ih[b] = pltpu.async_copy(
            inputs_hbm.at[pl.ds(base, CHUNK)], inp_v.at[b], sem_i.at[b])

    start_chunk(0)
    for c in range(NCHUNK):
        b = c % NBUF
        base = wid * B_PER_W + c * CHUNK
        if c + 1 < NCHUNK:
            nb = (c + 1) % NBUF
            if oh[nb] is not None:
                oh[nb].wait()
                th[nb].wait()
            start_chunk(c + 1)
        gh[b].wait()
        ih[b].wait()

        def body(i, carry, b=b):
            vg = buf_v[b, i, pl.ds(288, 16)]
            rot = _rotate4(inp_v[b, i, pl.ds(0, 16)])
            buf_v[b, i, pl.ds(288, 16)] = jnp.where(lanes < 12, vg, rot)
            for u in range(5):
                buf_v[b, i, pl.ds(304 + 16 * u, 16)] = \
                    inp_v[b, i, pl.ds(4 + 16 * u, 16)]
            tail_v[b, i, pl.ds(0, 16)] = inp_v[b, i, pl.ds(SPLIT, 16)]
            tail_v[b, i, pl.ds(28, 16)] = inp_v[b, i, pl.ds(112, 16)]
            tail_v[b, i, pl.ds(16, 16)] = inp_v[b, i, pl.ds(100, 16)]
            return carry
        lax.fori_loop(0, CHUNK, body, 0)

        oh[b] = pltpu.async_copy(
            buf_v.at[b],
            out_hbm.at[pl.ds(base, CHUNK), pl.ds(0, ROW_PAD)], sem_o.at[b])
        th[b] = pltpu.async_copy(
            tail_v.at[b],
            out_hbm.at[pl.ds(base, CHUNK), pl.ds(ROW_PAD, TAIL)], sem_t.at[b])

    for b in range(NBUF):
        oh[b].wait()
        th[b].wait()


def kernel(inputs, subject_id, s):
    return _concat_shape(inputs, subject_id.astype(jnp.int32), s)


# speedup vs baseline: 1.6662x; 1.0135x over previous
import functools
import jax
import jax.numpy as jnp
from jax import lax
from jax.experimental import pallas as pl
from jax.experimental.pallas import tpu as pltpu
from jax.experimental.pallas import tpu_sc as plsc

BATCH = 16384
FEAT = 128
SHAPE_DIM = 300
OUT_DIM = SHAPE_DIM + FEAT
ROW_PAD = 384
TAIL = OUT_DIM - ROW_PAD
SPLIT = ROW_PAD - SHAPE_DIM

NC = 2
NS = 16
NW = NC * NS
B_PER_W = BATCH // NW
CHUNK = 64
NCHUNK = B_PER_W // CHUNK
NBUF = 2

_mesh = plsc.VectorSubcoreMesh(core_axis_name="c", subcore_axis_name="s")


def _rotate4(v):
    idx = (lax.iota(jnp.int32, 16) + 4) % 16
    return lax.gather(
        v, idx[:, None],
        dimension_numbers=lax.GatherDimensionNumbers(
            offset_dims=(), collapsed_slice_dims=(0,), start_index_map=(0,)),
        slice_sizes=(1,),
        mode=lax.GatherScatterMode.PROMISE_IN_BOUNDS)


@functools.partial(
    pl.kernel,
    mesh=_mesh,
    out_type=jax.ShapeDtypeStruct((BATCH, OUT_DIM), jnp.float32),
    compiler_params=pltpu.CompilerParams(
        skip_device_barrier=True,
        disable_bounds_checks=True,
        disable_semaphore_checks=True,
    ),
    scratch_types=[
        pltpu.VMEM((B_PER_W,), jnp.int32),
        pltpu.VMEM((NBUF, CHUNK, ROW_PAD), jnp.float32),
        pltpu.VMEM((NBUF, CHUNK, FEAT), jnp.float32),
        pltpu.VMEM((NBUF, CHUNK, TAIL), jnp.float32),
        pltpu.SemaphoreType.DMA((NBUF,)),
        pltpu.SemaphoreType.DMA((NBUF,)),
        pltpu.SemaphoreType.DMA((NBUF,)),
        pltpu.SemaphoreType.DMA((NBUF,)),
    ],
)
def _concat_shape(inputs_hbm, idx_hbm, table_hbm, out_hbm,
                  idx_v, buf_v, inp_v, tail_v, sem_g, sem_i, sem_o, sem_t):
    wid = lax.axis_index("s") * NC + lax.axis_index("c")
    lanes = lax.iota(jnp.int32, 16)

    gh = [None] * NBUF
    ih = [None] * NBUF
    oh = [None] * NBUF
    th = [None] * NBUF

    pltpu.sync_copy(idx_hbm.at[pl.ds(wid * B_PER_W, B_PER_W)], idx_v)

    def start_chunk(c):
        b = c % NBUF
        base = wid * B_PER_W + c * CHUNK
        gh[b] = pltpu.async_copy(
            table_hbm.at[idx_v.at[pl.ds(c * CHUNK, CHUNK)], pl.ds(0, ROW_PAD)],
            buf_v.at[b], sem_g.at[b])
        ih[b] = pltpu.async_copy(
            inputs_hbm.at[pl.ds(base, CHUNK)], inp_v.at[b], sem_i.at[b])

    start_chunk(0)
    for c in range(NCHUNK):
        b = c % NBUF
        base = wid * B_PER_W + c * CHUNK
        if c + 1 < NCHUNK:
            nb = (c + 1) % NBUF
            if oh[nb] is not None:
                oh[nb].wait()
                th[nb].wait()
            start_chunk(c + 1)
        gh[b].wait()
        ih[b].wait()

        def body(i, carry, b=b):
            vg = buf_v[b, i, pl.ds(288, 16)]
            rot = _rotate4(inp_v[b, i, pl.ds(0, 16)])
            buf_v[b, i, pl.ds(288, 16)] = jnp.where(lanes < 12, vg, rot)
            for u in range(5):
                buf_v[b, i, pl.ds(304 + 16 * u, 16)] = \
                    inp_v[b, i, pl.ds(4 + 16 * u, 16)]
            tail_v[b, i, pl.ds(0, 16)] = inp_v[b, i, pl.ds(SPLIT, 16)]
            tail_v[b, i, pl.ds(28, 16)] = inp_v[b, i, pl.ds(112, 16)]
            tail_v[b, i, pl.ds(16, 16)] = inp_v[b, i, pl.ds(100, 16)]
            return carry
        lax.fori_loop(0, CHUNK, body, 0)

        oh[b] = pltpu.async_copy(
            buf_v.at[b],
            out_hbm.at[pl.ds(base, CHUNK), pl.ds(0, ROW_PAD)], sem_o.at[b])
        th[b] = pltpu.async_copy(
            tail_v.at[b],
            out_hbm.at[pl.ds(base, CHUNK), pl.ds(ROW_PAD, TAIL)], sem_t.at[b])

    for b in range(NBUF):
        oh[b].wait()
        th[b].wait()


def kernel(inputs, subject_id, s):
    return _concat_shape(inputs, subject_id.astype(jnp.int32), s)
